# baseline (speedup 1.0000x reference)
import jax
import jax.numpy as jnp
from jax.experimental import pallas as pl
from jax.experimental.pallas import tpu as pltpu

_H = 512
_W = 512
_CB = 8


def _dw3x3_kernel(w_ref, x_ref, o_ref):
    zrow = jnp.zeros((1, _W), jnp.float32)
    zcol = jnp.zeros((_H, 1), jnp.float32)
    for ch in range(_CB):
        c = pl.program_id(1) * _CB + ch
        x = x_ref[0, ch]
        xl = jnp.concatenate([zcol, x[:, :-1]], axis=1)
        xr = jnp.concatenate([x[:, 1:], zcol], axis=1)
        w = [w_ref[c, k] for k in range(9)]
        h0 = w[0] * xl + w[1] * x + w[2] * xr
        h1 = w[3] * xl + w[4] * x + w[5] * xr
        h2 = w[6] * xl + w[7] * x + w[8] * xr
        o_ref[0, ch] = (
            h1
            + jnp.concatenate([zrow, h0[:-1, :]], axis=0)
            + jnp.concatenate([h2[1:, :], zrow], axis=0)
        )


def _half_conv(x3, wmat, n_skip, n_take):
    skip_blk = n_skip // _CB

    def body(w_ref, x_ref, o_ref):
        zrow = jnp.zeros((1, _W), jnp.float32)
        zcol = jnp.zeros((_H, 1), jnp.float32)
        for ch in range(_CB):
            img = n_skip + pl.program_id(0) * _CB + ch
            c = img % 96
            x = x_ref[ch]
            xl = jnp.concatenate([zcol, x[:, :-1]], axis=1)
            xr = jnp.concatenate([x[:, 1:], zcol], axis=1)
            w = [w_ref[c, k] for k in range(9)]
            h0 = w[0] * xl + w[1] * x + w[2] * xr
            h1 = w[3] * xl + w[4] * x + w[5] * xr
            h2 = w[6] * xl + w[7] * x + w[8] * xr
            o_ref[ch] = (
                h1
                + jnp.concatenate([zrow, h0[:-1, :]], axis=0)
                + jnp.concatenate([h2[1:, :], zrow], axis=0)
            )

    return pl.pallas_call(
        body,
        grid=(n_take // _CB,),
        in_specs=[
            pl.BlockSpec(memory_space=pltpu.SMEM),
            pl.BlockSpec((_CB, _H, _W), lambda j: (j + skip_blk, 0, 0)),
        ],
        out_specs=pl.BlockSpec((_CB, _H, _W), lambda j: (j, 0, 0)),
        out_shape=jax.ShapeDtypeStruct((n_take, _H, _W), jnp.float32),
    )(wmat, x3)


def kernel(x, weight):
    n, ch, h, w = x.shape
    n_img = n * ch
    wmat = weight.reshape(ch, 9)
    x3 = x.reshape(n_img, h, w)
    a = _half_conv(x3, wmat, 0, n_img // 2)
    b = _half_conv(x3, wmat, n_img // 2, n_img // 2)
    return jnp.concatenate([a, b], axis=0).reshape(n, ch, h, w)


# speedup vs baseline: 1.7273x; 1.7273x over previous
import jax
import jax.numpy as jnp
from jax.experimental import pallas as pl
from jax.experimental.pallas import tpu as pltpu

_H = 512
_W = 512
_CB = 8


def _dw3x3_kernel(w_ref, x_ref, o_ref):
    zrow = jnp.zeros((1, _W), jnp.float32)
    zcol = jnp.zeros((_H, 1), jnp.float32)
    for ch in range(_CB):
        c = pl.program_id(1) * _CB + ch
        x = x_ref[0, ch]
        xl = jnp.concatenate([zcol, x[:, :-1]], axis=1)
        xr = jnp.concatenate([x[:, 1:], zcol], axis=1)
        w = [w_ref[c, k] for k in range(9)]
        h0 = w[0] * xl + w[1] * x + w[2] * xr
        h1 = w[3] * xl + w[4] * x + w[5] * xr
        h2 = w[6] * xl + w[7] * x + w[8] * xr
        o_ref[0, ch] = (
            h1
            + jnp.concatenate([zrow, h0[:-1, :]], axis=0)
            + jnp.concatenate([h2[1:, :], zrow], axis=0)
        )


def kernel(x, weight):
    n, ch, h, w = x.shape
    wmat = weight.reshape(ch, 9)
    grid = (n, ch // _CB)
    return pl.pallas_call(
        _dw3x3_kernel,
        grid=grid,
        in_specs=[
            pl.BlockSpec(memory_space=pltpu.SMEM),
            pl.BlockSpec((1, _CB, h, w), lambda i, j: (i, j, 0, 0)),
        ],
        out_specs=pl.BlockSpec((1, _CB, h, w), lambda i, j: (i, j, 0, 0)),
        out_shape=jax.ShapeDtypeStruct((n, ch, h, w), x.dtype),
    )(wmat, x)
